# baseline (speedup 1.0000x reference)
import functools

import jax
import jax.numpy as jnp
from jax import lax
from jax.experimental import pallas as pl
from jax.experimental.pallas import tpu as pltpu
from jax.experimental.pallas import tpu_sc as plsc


def _sc_gather_call(et0, et1, mul_tbl, bias_tbl, chunk, n_chunks, e_per_worker):
    E = et0.shape[0]
    T = mul_tbl.shape[0]
    mesh = plsc.VectorSubcoreMesh(core_axis_name="c", subcore_axis_name="s")

    @functools.partial(
        pl.kernel,
        mesh=mesh,
        compiler_params=pltpu.CompilerParams(needs_layout_passes=False),
        out_type=[
            jax.ShapeDtypeStruct((E,), jnp.float32),
            jax.ShapeDtypeStruct((E,), jnp.float32),
        ],
        scratch_types=[
            pltpu.VMEM((chunk,), jnp.int32),
            pltpu.VMEM((chunk,), jnp.int32),
            pltpu.VMEM((chunk,), jnp.int32),
            pltpu.VMEM((chunk,), jnp.int32),
            pltpu.VMEM((chunk,), jnp.float32),
            pltpu.VMEM((chunk,), jnp.float32),
            pltpu.VMEM((chunk,), jnp.float32),
            pltpu.VMEM((chunk,), jnp.float32),
            pltpu.VMEM((T,), jnp.float32),
            pltpu.VMEM((T,), jnp.float32),
            pltpu.SemaphoreType.DMA,
            pltpu.SemaphoreType.DMA,
            pltpu.SemaphoreType.DMA,
            pltpu.SemaphoreType.DMA,
            pltpu.SemaphoreType.DMA,
            pltpu.SemaphoreType.DMA,
            pltpu.SemaphoreType.DMA,
            pltpu.SemaphoreType.DMA,
        ],
    )
    def sc_kernel(et0_hbm, et1_hbm, mt_hbm, bt_hbm, mul_out, bias_out,
                  et0_a, et0_b, et1_a, et1_b, mul_a, mul_b, bias_a, bias_b,
                  mt_v, bt_v, is00, is01, is10, is11, os00, os01, os10, os11):
        nc = 2
        wid = lax.axis_index("s") * nc + lax.axis_index("c")
        pltpu.sync_copy(mt_hbm, mt_v)
        pltpu.sync_copy(bt_hbm, bt_v)
        base_e = wid * e_per_worker
        et0_bufs = (et0_a, et0_b)
        et1_bufs = (et1_a, et1_b)
        mul_bufs = (mul_a, mul_b)
        bias_bufs = (bias_a, bias_b)
        in_sems = ((is00, is01), (is10, is11))
        out_sems = ((os00, os01), (os10, os11))

        def start_in(ci):
            b = ci % 2
            e0 = base_e + ci * chunk
            return (
                pltpu.async_copy(
                    et0_hbm.at[pl.ds(e0, chunk)], et0_bufs[b], in_sems[b][0]),
                pltpu.async_copy(
                    et1_hbm.at[pl.ds(e0, chunk)], et1_bufs[b], in_sems[b][1]),
            )

        def start_out(ci):
            b = ci % 2
            e0 = base_e + ci * chunk
            return (
                pltpu.async_copy(
                    mul_bufs[b], mul_out.at[pl.ds(e0, chunk)], out_sems[b][0]),
                pltpu.async_copy(
                    bias_bufs[b], bias_out.at[pl.ds(e0, chunk)], out_sems[b][1]),
            )

        pend_in = start_in(0)
        pend_out = [None, None]
        for ci in range(n_chunks):
            b = ci % 2
            nxt = start_in(ci + 1) if ci + 1 < n_chunks else None
            for h in pend_in:
                h.wait()
            if pend_out[b] is not None:
                for h in pend_out[b]:
                    h.wait()
                pend_out[b] = None
            et0_v, et1_v = et0_bufs[b], et1_bufs[b]
            mul_v, bias_v = mul_bufs[b], bias_bufs[b]

            def grp(j, c2):
                o = j * 16
                i0 = et0_v[pl.ds(o, 16)]
                i1 = et1_v[pl.ds(o, 16)]
                mul_v[pl.ds(o, 16)] = (
                    plsc.load_gather(mt_v, [i0]) + plsc.load_gather(mt_v, [i1]))
                bias_v[pl.ds(o, 16)] = (
                    plsc.load_gather(bt_v, [i0]) + plsc.load_gather(bt_v, [i1]))
                return c2

            lax.fori_loop(0, chunk // 16, grp, 0, unroll=4)
            pend_out[b] = start_out(ci)
            if nxt is not None:
                pend_in = nxt
        for po in pend_out:
            if po is not None:
                for h in po:
                    h.wait()

    return sc_kernel(et0, et1, mul_tbl, bias_tbl)


_SIN_C = (6.283182793407033, -41.34141938561704, 81.59613875538135,
          -76.5796878510129, 41.203743633642276, -12.268859940984608)


_SUB = 512


def _tc_body(x_ref, m_ref, b_ref, w_ref, o_ref):
    wr = w_ref[...]
    n_sub = o_ref.shape[1] // _SUB

    def step(i, carry):
        sl = pl.ds(i * _SUB, _SUB)
        xb = x_ref[:, sl]
        coef = m_ref[:, sl] / xb
        r = wr * xb
        t = r - jnp.floor(r + 0.5)
        u = t * t
        p = jnp.float32(_SIN_C[5])
        for c in (4, 3, 2, 1, 0):
            p = p * u + jnp.float32(_SIN_C[c])
        o_ref[:, sl] = coef * (p * t) + b_ref[:, sl]
        return carry

    lax.fori_loop(0, n_sub, step, 0, unroll=5)


def _tc_basis_call(x2, mul2, bias2, w2, block_cols):
    nb = w2.shape[0]
    E = x2.shape[1]
    grid = (E // block_cols,)
    return pl.pallas_call(
        _tc_body,
        grid=grid,
        in_specs=[
            pl.BlockSpec((1, block_cols), lambda i: (0, i)),
            pl.BlockSpec((1, block_cols), lambda i: (0, i)),
            pl.BlockSpec((1, block_cols), lambda i: (0, i)),
            pl.BlockSpec((nb, 1), lambda i: (0, 0)),
        ],
        out_specs=pl.BlockSpec((nb, block_cols), lambda i: (0, i)),
        out_shape=jax.ShapeDtypeStruct((nb, E), jnp.float32),
        compiler_params=pltpu.CompilerParams(
            dimension_semantics=("arbitrary",),
        ),
    )(x2, mul2, bias2, w2)


def kernel(x, edge_types, mul_weight, bias_weight, bessel_weights, prefactor):
    E = x.shape[0]
    nb = bessel_weights.shape[0]

    mul_tbl = mul_weight[:, 0] * prefactor
    bias_tbl = bias_weight[:, 0]
    et0 = edge_types[:, 0]
    et1 = edge_types[:, 1]

    n_workers = 32
    e_per_worker = E // n_workers
    chunk = 10000
    n_chunks = e_per_worker // chunk
    mul_e, bias_e = _sc_gather_call(
        et0, et1, mul_tbl, bias_tbl, chunk, n_chunks, e_per_worker)

    x2 = x.reshape(1, E)
    m2 = mul_e.reshape(1, E)
    b2 = bias_e.reshape(1, E)
    w2 = (bessel_weights * jnp.float32(1.0 / (2.0 * jnp.pi))).reshape(nb, 1)

    out_t = _tc_basis_call(x2, m2, b2, w2, block_cols=12800)
    return out_t.T


# speedup vs baseline: 1.1619x; 1.1619x over previous
import functools

import jax
import jax.numpy as jnp
from jax import lax
from jax.experimental import pallas as pl
from jax.experimental.pallas import tpu as pltpu
from jax.experimental.pallas import tpu_sc as plsc


def _sc_gather_call(et0, et1, mul_tbl, bias_tbl, chunk, n_chunks, e_per_worker):
    E = et0.shape[0]
    T = mul_tbl.shape[0]
    mesh = plsc.VectorSubcoreMesh(core_axis_name="c", subcore_axis_name="s")

    @functools.partial(
        pl.kernel,
        mesh=mesh,
        compiler_params=pltpu.CompilerParams(needs_layout_passes=False),
        out_type=[
            jax.ShapeDtypeStruct((E,), jnp.float32),
            jax.ShapeDtypeStruct((E,), jnp.float32),
        ],
        scratch_types=[
            pltpu.VMEM((chunk,), jnp.int32),
            pltpu.VMEM((chunk,), jnp.int32),
            pltpu.VMEM((chunk,), jnp.int32),
            pltpu.VMEM((chunk,), jnp.int32),
            pltpu.VMEM((chunk,), jnp.float32),
            pltpu.VMEM((chunk,), jnp.float32),
            pltpu.VMEM((chunk,), jnp.float32),
            pltpu.VMEM((chunk,), jnp.float32),
            pltpu.VMEM((T,), jnp.float32),
            pltpu.VMEM((T,), jnp.float32),
            pltpu.SemaphoreType.DMA,
            pltpu.SemaphoreType.DMA,
            pltpu.SemaphoreType.DMA,
            pltpu.SemaphoreType.DMA,
            pltpu.SemaphoreType.DMA,
            pltpu.SemaphoreType.DMA,
            pltpu.SemaphoreType.DMA,
            pltpu.SemaphoreType.DMA,
        ],
    )
    def sc_kernel(et0_hbm, et1_hbm, mt_hbm, bt_hbm, mul_out, bias_out,
                  et0_a, et0_b, et1_a, et1_b, mul_a, mul_b, bias_a, bias_b,
                  mt_v, bt_v, is00, is01, is10, is11, os00, os01, os10, os11):
        nc = 2
        wid = lax.axis_index("s") * nc + lax.axis_index("c")
        pltpu.sync_copy(mt_hbm, mt_v)
        pltpu.sync_copy(bt_hbm, bt_v)
        base_e = wid * e_per_worker
        et0_bufs = (et0_a, et0_b)
        et1_bufs = (et1_a, et1_b)
        mul_bufs = (mul_a, mul_b)
        bias_bufs = (bias_a, bias_b)
        in_sems = ((is00, is01), (is10, is11))
        out_sems = ((os00, os01), (os10, os11))

        def start_in(ci):
            b = ci % 2
            e0 = base_e + ci * chunk
            return (
                pltpu.async_copy(
                    et0_hbm.at[pl.ds(e0, chunk)], et0_bufs[b], in_sems[b][0]),
                pltpu.async_copy(
                    et1_hbm.at[pl.ds(e0, chunk)], et1_bufs[b], in_sems[b][1]),
            )

        def start_out(ci):
            b = ci % 2
            e0 = base_e + ci * chunk
            return (
                pltpu.async_copy(
                    mul_bufs[b], mul_out.at[pl.ds(e0, chunk)], out_sems[b][0]),
                pltpu.async_copy(
                    bias_bufs[b], bias_out.at[pl.ds(e0, chunk)], out_sems[b][1]),
            )

        pend_in = start_in(0)
        pend_out = [None, None]
        for ci in range(n_chunks):
            b = ci % 2
            nxt = start_in(ci + 1) if ci + 1 < n_chunks else None
            for h in pend_in:
                h.wait()
            if pend_out[b] is not None:
                for h in pend_out[b]:
                    h.wait()
                pend_out[b] = None
            et0_v, et1_v = et0_bufs[b], et1_bufs[b]
            mul_v, bias_v = mul_bufs[b], bias_bufs[b]

            def grp(j, c2):
                o = j * 16
                i0 = et0_v[pl.ds(o, 16)]
                i1 = et1_v[pl.ds(o, 16)]
                mul_v[pl.ds(o, 16)] = (
                    plsc.load_gather(mt_v, [i0]) + plsc.load_gather(mt_v, [i1]))
                bias_v[pl.ds(o, 16)] = (
                    plsc.load_gather(bt_v, [i0]) + plsc.load_gather(bt_v, [i1]))
                return c2

            lax.fori_loop(0, chunk // 16, grp, 0, unroll=4)
            pend_out[b] = start_out(ci)
            if nxt is not None:
                pend_in = nxt
        for po in pend_out:
            if po is not None:
                for h in po:
                    h.wait()

    return sc_kernel(et0, et1, mul_tbl, bias_tbl)


_SIN_C = (6.283182793407033, -41.34141938561704, 81.59613875538135,
          -76.5796878510129, 41.203743633642276, -12.268859940984608)


_SUB = 512


def _tc_body(x_ref, m_ref, b_ref, w_ref, o_ref):
    wr = w_ref[...]
    n_sub = o_ref.shape[1] // _SUB

    def step(i, carry):
        sl = pl.ds(i * _SUB, _SUB)
        xb = x_ref[:, sl]
        coef = m_ref[:, sl] / xb
        r = wr * xb
        t = r - jnp.floor(r + 0.5)
        u = t * t
        p = jnp.float32(_SIN_C[5])
        for c in (4, 3, 2, 1, 0):
            p = p * u + jnp.float32(_SIN_C[c])
        o_ref[:, sl] = coef * (p * t) + b_ref[:, sl]
        return carry

    lax.fori_loop(0, n_sub, step, 0, unroll=5)


def _tc_body_prev(x_ref, m_ref, b_ref, w_ref, prev_ref, o_ref):
    del prev_ref
    _tc_body(x_ref, m_ref, b_ref, w_ref, o_ref)


def _tc_basis_call(x2, mul2, bias2, w2, block_cols, e_total, block_off, prev):
    nb = w2.shape[0]
    e_part = x2.shape[1]
    grid = (e_part // block_cols,)
    in_specs = [
        pl.BlockSpec((1, block_cols), lambda i: (0, i)),
        pl.BlockSpec((1, block_cols), lambda i: (0, i)),
        pl.BlockSpec((1, block_cols), lambda i: (0, i)),
        pl.BlockSpec((nb, 1), lambda i: (0, 0)),
    ]
    operands = [x2, mul2, bias2, w2]
    body = _tc_body
    io_alias = {}
    if prev is not None:
        in_specs.append(pl.BlockSpec(memory_space=pl.ANY))
        operands.append(prev)
        io_alias = {4: 0}
        body = _tc_body_prev
    return pl.pallas_call(
        body,
        grid=grid,
        in_specs=in_specs,
        out_specs=pl.BlockSpec((nb, block_cols), lambda i: (0, i + block_off)),
        out_shape=jax.ShapeDtypeStruct((nb, e_total), jnp.float32),
        input_output_aliases=io_alias,
        compiler_params=pltpu.CompilerParams(
            dimension_semantics=("arbitrary",),
        ),
    )(*operands)


def kernel(x, edge_types, mul_weight, bias_weight, bessel_weights, prefactor):
    E = x.shape[0]
    nb = bessel_weights.shape[0]

    mul_tbl = mul_weight[:, 0] * prefactor
    bias_tbl = bias_weight[:, 0]
    et0 = edge_types[:, 0]
    et1 = edge_types[:, 1]

    e_a = 576000
    e_b = E - e_a
    mul_a, bias_a = _sc_gather_call(
        et0[:e_a], et1[:e_a], mul_tbl, bias_tbl, 6000, 3, 18000)
    mul_b, bias_b = _sc_gather_call(
        et0[e_a:], et1[e_a:], mul_tbl, bias_tbl, 8000, 4, 32000)

    w2 = (bessel_weights * jnp.float32(1.0 / (2.0 * jnp.pi))).reshape(nb, 1)
    bc = 12800
    out_a = _tc_basis_call(
        x[:e_a].reshape(1, e_a), mul_a.reshape(1, e_a),
        bias_a.reshape(1, e_a), w2, bc, E, 0, None)
    out_t = _tc_basis_call(
        x[e_a:].reshape(1, e_b), mul_b.reshape(1, e_b),
        bias_b.reshape(1, e_b), w2, bc, E, e_a // bc, out_a)
    return out_t.T


# speedup vs baseline: 1.1991x; 1.0320x over previous
import functools

import jax
import jax.numpy as jnp
from jax import lax
from jax.experimental import pallas as pl
from jax.experimental.pallas import tpu as pltpu
from jax.experimental.pallas import tpu_sc as plsc


def _sc_gather_call(et0, et1, mul_tbl, bias_tbl, chunk, n_chunks, e_per_worker):
    E = et0.shape[0]
    T = mul_tbl.shape[0]
    mesh = plsc.VectorSubcoreMesh(core_axis_name="c", subcore_axis_name="s")

    @functools.partial(
        pl.kernel,
        mesh=mesh,
        compiler_params=pltpu.CompilerParams(needs_layout_passes=False),
        out_type=[
            jax.ShapeDtypeStruct((E,), jnp.float32),
            jax.ShapeDtypeStruct((E,), jnp.float32),
        ],
        scratch_types=[
            pltpu.VMEM((chunk,), jnp.int32),
            pltpu.VMEM((chunk,), jnp.int32),
            pltpu.VMEM((chunk,), jnp.int32),
            pltpu.VMEM((chunk,), jnp.int32),
            pltpu.VMEM((chunk,), jnp.float32),
            pltpu.VMEM((chunk,), jnp.float32),
            pltpu.VMEM((chunk,), jnp.float32),
            pltpu.VMEM((chunk,), jnp.float32),
            pltpu.VMEM((T,), jnp.float32),
            pltpu.VMEM((T,), jnp.float32),
            pltpu.SemaphoreType.DMA,
            pltpu.SemaphoreType.DMA,
            pltpu.SemaphoreType.DMA,
            pltpu.SemaphoreType.DMA,
            pltpu.SemaphoreType.DMA,
            pltpu.SemaphoreType.DMA,
            pltpu.SemaphoreType.DMA,
            pltpu.SemaphoreType.DMA,
        ],
    )
    def sc_kernel(et0_hbm, et1_hbm, mt_hbm, bt_hbm, mul_out, bias_out,
                  et0_a, et0_b, et1_a, et1_b, mul_a, mul_b, bias_a, bias_b,
                  mt_v, bt_v, is00, is01, is10, is11, os00, os01, os10, os11):
        nc = 2
        wid = lax.axis_index("s") * nc + lax.axis_index("c")
        pltpu.sync_copy(mt_hbm, mt_v)
        pltpu.sync_copy(bt_hbm, bt_v)
        base_e = wid * e_per_worker
        et0_bufs = (et0_a, et0_b)
        et1_bufs = (et1_a, et1_b)
        mul_bufs = (mul_a, mul_b)
        bias_bufs = (bias_a, bias_b)
        in_sems = ((is00, is01), (is10, is11))
        out_sems = ((os00, os01), (os10, os11))

        def start_in(ci):
            b = ci % 2
            e0 = base_e + ci * chunk
            return (
                pltpu.async_copy(
                    et0_hbm.at[pl.ds(e0, chunk)], et0_bufs[b], in_sems[b][0]),
                pltpu.async_copy(
                    et1_hbm.at[pl.ds(e0, chunk)], et1_bufs[b], in_sems[b][1]),
            )

        def start_out(ci):
            b = ci % 2
            e0 = base_e + ci * chunk
            return (
                pltpu.async_copy(
                    mul_bufs[b], mul_out.at[pl.ds(e0, chunk)], out_sems[b][0]),
                pltpu.async_copy(
                    bias_bufs[b], bias_out.at[pl.ds(e0, chunk)], out_sems[b][1]),
            )

        pend_in = start_in(0)
        pend_out = [None, None]
        for ci in range(n_chunks):
            b = ci % 2
            nxt = start_in(ci + 1) if ci + 1 < n_chunks else None
            for h in pend_in:
                h.wait()
            if pend_out[b] is not None:
                for h in pend_out[b]:
                    h.wait()
                pend_out[b] = None
            et0_v, et1_v = et0_bufs[b], et1_bufs[b]
            mul_v, bias_v = mul_bufs[b], bias_bufs[b]

            def grp(j, c2):
                o = j * 16
                i0 = et0_v[pl.ds(o, 16)]
                i1 = et1_v[pl.ds(o, 16)]
                mul_v[pl.ds(o, 16)] = (
                    plsc.load_gather(mt_v, [i0]) + plsc.load_gather(mt_v, [i1]))
                bias_v[pl.ds(o, 16)] = (
                    plsc.load_gather(bt_v, [i0]) + plsc.load_gather(bt_v, [i1]))
                return c2

            lax.fori_loop(0, chunk // 16, grp, 0, unroll=4)
            pend_out[b] = start_out(ci)
            if nxt is not None:
                pend_in = nxt
        for po in pend_out:
            if po is not None:
                for h in po:
                    h.wait()

    return sc_kernel(et0, et1, mul_tbl, bias_tbl)


_SIN_C = (6.283182793407033, -41.34141938561704, 81.59613875538135,
          -76.5796878510129, 41.203743633642276, -12.268859940984608)


_SUB = 512


def _tc_body(x_ref, m_ref, b_ref, w_ref, o_ref):
    wr = w_ref[...]
    n_sub = o_ref.shape[1] // _SUB

    def step(i, carry):
        sl = pl.ds(i * _SUB, _SUB)
        xb = x_ref[:, sl]
        coef = m_ref[:, sl] / xb
        r = wr * xb
        t = r - jnp.floor(r + 0.5)
        u = t * t
        p = jnp.float32(_SIN_C[5])
        for c in (4, 3, 2, 1, 0):
            p = p * u + jnp.float32(_SIN_C[c])
        o_ref[:, sl] = coef * (p * t) + b_ref[:, sl]
        return carry

    lax.fori_loop(0, n_sub, step, 0, unroll=5)


def _tc_body_prev(x_ref, m_ref, b_ref, w_ref, prev_ref, o_ref):
    del prev_ref
    _tc_body(x_ref, m_ref, b_ref, w_ref, o_ref)


def _tc_basis_call(x2, mul2, bias2, w2, block_cols, e_total, block_off, prev):
    nb = w2.shape[0]
    e_part = x2.shape[1]
    grid = (e_part // block_cols,)
    in_specs = [
        pl.BlockSpec((1, block_cols), lambda i: (0, i)),
        pl.BlockSpec((1, block_cols), lambda i: (0, i)),
        pl.BlockSpec((1, block_cols), lambda i: (0, i)),
        pl.BlockSpec((nb, 1), lambda i: (0, 0)),
    ]
    operands = [x2, mul2, bias2, w2]
    body = _tc_body
    io_alias = {}
    if prev is not None:
        in_specs.append(pl.BlockSpec(memory_space=pl.ANY))
        operands.append(prev)
        io_alias = {4: 0}
        body = _tc_body_prev
    return pl.pallas_call(
        body,
        grid=grid,
        in_specs=in_specs,
        out_specs=pl.BlockSpec((nb, block_cols), lambda i: (0, i + block_off)),
        out_shape=jax.ShapeDtypeStruct((nb, e_total), jnp.float32),
        input_output_aliases=io_alias,
        compiler_params=pltpu.CompilerParams(
            dimension_semantics=("arbitrary",),
        ),
    )(*operands)


def kernel(x, edge_types, mul_weight, bias_weight, bessel_weights, prefactor):
    E = x.shape[0]
    nb = bessel_weights.shape[0]

    mul_tbl = mul_weight[:, 0] * prefactor
    bias_tbl = bias_weight[:, 0]
    et0 = edge_types[:, 0]
    et1 = edge_types[:, 1]

    slices = (
        (294400, 9200, 1, 9200),
        (486400, 7600, 2, 15200),
        (819200, 6400, 4, 25600),
    )
    w2 = (bessel_weights * jnp.float32(1.0 / (2.0 * jnp.pi))).reshape(nb, 1)
    bc = 12800

    sc_outs = []
    e0 = 0
    for e_s, chunk, n_chunks, epw in slices:
        sc_outs.append(_sc_gather_call(
            et0[e0:e0 + e_s], et1[e0:e0 + e_s], mul_tbl, bias_tbl,
            chunk, n_chunks, epw))
        e0 += e_s

    out_t = None
    e0 = 0
    for (e_s, _, _, _), (mul_s, bias_s) in zip(slices, sc_outs):
        out_t = _tc_basis_call(
            x[e0:e0 + e_s].reshape(1, e_s), mul_s.reshape(1, e_s),
            bias_s.reshape(1, e_s), w2, bc, E, e0 // bc, out_t)
        e0 += e_s
    return out_t.T
